# baseline (speedup 1.0000x reference)
import functools

import jax
import jax.numpy as jnp
from jax import lax
from jax.experimental import pallas as pl
from jax.experimental.pallas import tpu as pltpu
from jax.experimental.pallas import tpu_sc as plsc

NW = 32
NBUF = 8


def _gather_kernel(per_w, x_hbm, table_hbm, out_hbm,
                   idx_v, rows_v, gsems, osems):
    nc = 2
    wid = lax.axis_index("s") * nc + lax.axis_index("c")
    s0 = wid * per_w
    pltpu.sync_copy(x_hbm.at[pl.ds(s0, per_w)], idx_v)

    @pl.loop(0, per_w, step=NBUF)
    def group(i0):
        descs = []
        for b in range(NBUF):
            @pl.when(i0 > 0)
            def _(b=b):
                pltpu.make_async_copy(
                    rows_v.at[b], out_hbm.at[0], osems[b]
                ).wait()
            descs.append(
                pltpu.async_copy(
                    table_hbm.at[idx_v.at[i0 + b]], rows_v.at[b], gsems[b]
                )
            )
        for b in range(NBUF):
            descs[b].wait()
            pltpu.async_copy(
                rows_v.at[b], out_hbm.at[s0 + i0 + b], osems[b]
            )

    for b in range(NBUF):
        pltpu.make_async_copy(
            rows_v.at[b], out_hbm.at[0], osems[b]
        ).wait()


def kernel(x, table):
    B, H = x.shape
    V, D = table.shape
    per_w = B // NW
    assert per_w * NW == B and per_w % NBUF == 0

    mesh = plsc.VectorSubcoreMesh(core_axis_name="c", subcore_axis_name="s")
    run = pl.kernel(
        functools.partial(_gather_kernel, per_w),
        out_type=jax.ShapeDtypeStruct((B, H, D), jnp.float32),
        mesh=mesh,
        scratch_types=[
            pltpu.VMEM((per_w, H), jnp.int32),
            pltpu.VMEM((NBUF, H, D), jnp.float32),
            [pltpu.SemaphoreType.DMA] * NBUF,
            [pltpu.SemaphoreType.DMA] * NBUF,
        ],
        compiler_params=pltpu.CompilerParams(use_tc_tiling_on_sc=False),
    )
    return run(x.astype(jnp.int32), table)
